# baseline (speedup 1.0000x reference)
import functools

import jax
import jax.numpy as jnp
from jax.experimental import pallas as pl
from jax.experimental.pallas import tpu as pltpu
from jax.experimental.pallas import tpu_sc as plsc

_B = 1024
_D = 64
_N = 100000
_K = 3
_L = 2
_NUM_LABELS = 12
_RATIO = 0.5
_SCALE = 4.0

_C = 2048
_NSTEPS = (_N + _C - 1) // _C
_NPAD = _NSTEPS * _C
_G = 128

_NEG = float("-inf")
_BIGI = 2**30


_RB = 64


def _knn_body(q_ref, db_ref, v0, v1, v2, i0, i1, i2, qn_s):
    t = pl.program_id(0)

    @pl.when(t == 0)
    def _init():
        q = q_ref[...]
        qn_s[...] = q / jnp.sqrt(jnp.sum(q * q, axis=1, keepdims=True))
        for vp in (v0, v1, v2):
            vp[...] = jnp.full((_B, _G), _NEG, jnp.float32)
        for ip in (i0, i1, i2):
            ip[...] = jnp.zeros((_B, _G), jnp.int32)

    f = db_ref[:, :_D]
    fn = f / jnp.sqrt(jnp.sum(f * f, axis=1, keepdims=True))
    s_full = jax.lax.dot_general(
        qn_s[...].astype(jnp.bfloat16), fn.astype(jnp.bfloat16),
        (((1,), (1,)), ((), ())),
        preferred_element_type=jnp.float32)

    lane = jax.lax.broadcasted_iota(jnp.int32, (_RB, _G), 1)
    for rb in range(_B // _RB):
        rs = pl.ds(rb * _RB, _RB)
        V0, V1, V2 = v0[rs, :], v1[rs, :], v2[rs, :]
        I0, I1, I2 = i0[rs, :], i1[rs, :], i2[rs, :]
        for qq in range(_C // _G):
            s = s_full[rb * _RB:(rb + 1) * _RB, qq * _G:(qq + 1) * _G]
            cc = t * (_C // _G) + qq
            s = jnp.where(lane < _N - cc * _G, s, _NEG)
            gt0 = s > V0
            gt1 = s > V1
            gt2 = s > V2
            V2 = jnp.where(gt1, V1, jnp.where(gt2, s, V2))
            I2 = jnp.where(gt1, I1, jnp.where(gt2, cc, I2))
            V1 = jnp.where(gt0, V0, jnp.where(gt1, s, V1))
            I1 = jnp.where(gt0, I0, jnp.where(gt1, cc, I1))
            V0 = jnp.where(gt0, s, V0)
            I0 = jnp.where(gt0, cc, I0)
        v0[rs, :], v1[rs, :], v2[rs, :] = V0, V1, V2
        i0[rs, :], i1[rs, :], i2[rs, :] = I0, I1, I2


def _knn_call(queries, database):
    plane_spec = pl.BlockSpec((_B, _G), lambda t: (0, 0))
    return pl.pallas_call(
        _knn_body,
        grid=(_NSTEPS,),
        in_specs=[
            pl.BlockSpec((_B, _D), lambda t: (0, 0)),
            pl.BlockSpec((_C, _D + 1), lambda t: (t, 0)),
        ],
        out_specs=[plane_spec] * 6,
        out_shape=[
            jax.ShapeDtypeStruct((_B, _G), jnp.float32),
            jax.ShapeDtypeStruct((_B, _G), jnp.float32),
            jax.ShapeDtypeStruct((_B, _G), jnp.float32),
            jax.ShapeDtypeStruct((_B, _G), jnp.int32),
            jax.ShapeDtypeStruct((_B, _G), jnp.int32),
            jax.ShapeDtypeStruct((_B, _G), jnp.int32),
        ],
        scratch_shapes=[
            pltpu.VMEM((_B, _D), jnp.float32),
        ],
    )(queries, database)


def _extract_body(v0, v1, v2, i0, i1, i2, scores_ref, idx_ref):
    allv = jnp.concatenate([v0[...], v1[...], v2[...]], axis=1)
    allc = jnp.concatenate([i0[...], i1[...], i2[...]], axis=1)
    lane = jax.lax.broadcasted_iota(jnp.int32, (_B, 3 * _G), 1) & (_G - 1)
    alli = allc * _G + lane
    av, ai = allv, alli
    for j in range(_K):
        m = jnp.max(av, axis=1, keepdims=True)
        hit = av == m
        cand = jnp.min(jnp.where(hit, ai, _BIGI), axis=1, keepdims=True)
        scores_ref[:, j:j + 1] = m
        idx_ref[:, j:j + 1] = cand
        av = jnp.where(hit & (ai == cand), _NEG, av)


def _extract_call(planes):
    return pl.pallas_call(
        _extract_body,
        out_shape=[
            jax.ShapeDtypeStruct((_B, _K), jnp.float32),
            jax.ShapeDtypeStruct((_B, _K), jnp.int32),
        ],
    )(*planes)


_NW = 32
_BPW = (_B * _K) // _NW


def _gather_body(table_hbm, idx_hbm, rows_out, idx_v, rows_v, sem):
    wid = jax.lax.axis_index("s") * 2 + jax.lax.axis_index("c")
    base = wid * _BPW
    pltpu.sync_copy(idx_hbm.at[pl.ds(base, _BPW)], idx_v)
    pltpu.async_copy(table_hbm.at[idx_v], rows_v, sem).wait()
    pltpu.sync_copy(rows_v, rows_out.at[pl.ds(base, _BPW)])


def _gather_call(table, idx_flat):
    fn = pl.kernel(
        _gather_body,
        mesh=plsc.VectorSubcoreMesh(core_axis_name="c", subcore_axis_name="s"),
        out_type=jax.ShapeDtypeStruct((_B * _K, 128), jnp.float32),
        scratch_types=[
            pltpu.VMEM((_BPW,), jnp.int32),
            pltpu.VMEM((_BPW, 128), jnp.float32),
            pltpu.SemaphoreType.DMA,
        ],
    )
    return fn(table, idx_flat)


def _proj(x, w, b):
    y = jax.lax.dot_general(x, w, (((1,), (1,)), ((), ())),
                            preferred_element_type=jnp.float32)
    return y + b.reshape(1, -1)


def _mea_body(q_ref, rows_ref, sc_ref,
              WQ_ref, bQ_ref, WK_ref, bK_ref, WV_ref, bV_ref, WO_ref, bO_ref,
              dW_ref, db_ref, oW_ref, ob_ref, out_ref):
    q = q_ref[...]
    rows = rows_ref[:, :_D]
    lab = rows_ref[:, _D:_D + 1]
    sc = sc_ref[...]

    rn = rows / jnp.sqrt(jnp.sum(rows * rows, axis=1, keepdims=True))
    labi = lab.astype(jnp.int32)
    lane = jax.lax.broadcasted_iota(jnp.int32, (_B * _K, _D), 1)
    sc_km = jnp.concatenate([sc[:, k:k + 1] for k in range(_K)], axis=0)
    cls = jnp.where(lane == labi, sc_km, jnp.float32(0.0))
    hx = jnp.concatenate([q, q, q], axis=0)

    T = [cls, hx, rn]
    for i in range(_L):
        Q = [_proj(T[j], WQ_ref[i], bQ_ref[i]) for j in range(3)]
        Kt = [_proj(T[j], WK_ref[i], bK_ref[i]) for j in range(3)]
        V = [_proj(T[j], WV_ref[i], bV_ref[i]) for j in range(3)]
        newT = []
        for a in range(3):
            att = [jnp.sum(Q[a] * Kt[b], axis=1, keepdims=True) / _SCALE
                   for b in range(3)]
            m = jnp.maximum(jnp.maximum(att[0], att[1]), att[2])
            e = [jnp.exp(x - m) for x in att]
            den = e[0] + e[1] + e[2]
            o = (e[0] * V[0] + e[1] * V[1] + e[2] * V[2]) / den
            newT.append(_proj(o, WO_ref[i], bO_ref[i]))
        T = newT

    x = jnp.tanh(_proj(T[0], dW_ref[...], db_ref[...]))
    logits = _proj(x, oW_ref[...], ob_ref[...])
    lm = (logits[0:_B] + logits[_B:2 * _B] + logits[2 * _B:3 * _B]) / 3.0

    lane12 = jax.lax.broadcasted_iota(jnp.int32, (_B, _NUM_LABELS), 1)
    oh = jnp.zeros((_B, _NUM_LABELS), jnp.float32)
    for k in range(_K):
        oh = oh + jnp.where(lane12 == labi[k * _B:(k + 1) * _B], 1.0, 0.0)
    retr = oh / jnp.sum(oh, axis=1, keepdims=True)

    out_ref[...] = lm * (1.0 - _RATIO) + retr * _RATIO


def _mea_call(queries, rows128, scores,
              WQ, bQ, WK, bK, WV, bV, WO, bO, dense_W, dense_b, out_W, out_b):
    return pl.pallas_call(
        _mea_body,
        out_shape=jax.ShapeDtypeStruct((_B, _NUM_LABELS), jnp.float32),
    )(queries, rows128, scores,
      WQ, bQ, WK, bK, WV, bV, WO, bO, dense_W, dense_b, out_W, out_b)


def kernel(queries, database, WQ, bQ, WK, bK, WV, bV, WO, bO,
           dense_W, dense_b, out_W, out_b):
    planes = _knn_call(queries, database)
    scores, idxs = _extract_call(planes)
    idx_km = idxs.T.reshape(-1)
    table = jnp.pad(database, ((0, 0), (0, 128 - (_D + 1))))
    rows128 = _gather_call(table, idx_km)
    return _mea_call(queries, rows128, scores,
                     WQ, bQ, WK, bK, WV, bV, WO, bO,
                     dense_W, dense_b, out_W, out_b)


# speedup vs baseline: 1.0137x; 1.0137x over previous
import functools

import jax
import jax.numpy as jnp
from jax.experimental import pallas as pl
from jax.experimental.pallas import tpu as pltpu
from jax.experimental.pallas import tpu_sc as plsc

_B = 1024
_D = 64
_N = 100000
_K = 3
_L = 2
_NUM_LABELS = 12
_RATIO = 0.5
_SCALE = 4.0

_C = 2048
_NSTEPS = (_N + _C - 1) // _C
_NPAD = _NSTEPS * _C
_G = 128

_NEG = float("-inf")
_BIGI = 2**30


def _knn_body(q_ref, db_ref, scores_ref, idx_ref, qn_s, v0, v1, v2, i0, i1, i2):
    t = pl.program_id(0)

    @pl.when(t == 0)
    def _init():
        q = q_ref[...]
        qn_s[...] = q / jnp.sqrt(jnp.sum(q * q, axis=1, keepdims=True))
        for vp in (v0, v1, v2):
            vp[...] = jnp.full((_B, _G), _NEG, jnp.float32)
        for ip in (i0, i1, i2):
            ip[...] = jnp.zeros((_B, _G), jnp.int32)
        scores_ref[...] = jnp.zeros((_B, _K), jnp.float32)
        idx_ref[...] = jnp.zeros((_B, _K), jnp.int32)

    f = db_ref[:, :_D]
    fn = f / jnp.sqrt(jnp.sum(f * f, axis=1, keepdims=True))
    s_full = jax.lax.dot_general(
        qn_s[...], fn, (((1,), (1,)), ((), ())),
        preferred_element_type=jnp.float32)

    lane = jax.lax.broadcasted_iota(jnp.int32, (_B, _G), 1)
    V0, V1, V2 = v0[...], v1[...], v2[...]
    I0, I1, I2 = i0[...], i1[...], i2[...]
    for qq in range(_C // _G):
        s = s_full[:, qq * _G:(qq + 1) * _G]
        cc = t * (_C // _G) + qq
        s = jnp.where(lane < _N - cc * _G, s, _NEG)
        gt0 = s > V0
        gt1 = s > V1
        gt2 = s > V2
        V2 = jnp.where(gt1, V1, jnp.where(gt2, s, V2))
        I2 = jnp.where(gt1, I1, jnp.where(gt2, cc, I2))
        V1 = jnp.where(gt0, V0, jnp.where(gt1, s, V1))
        I1 = jnp.where(gt0, I0, jnp.where(gt1, cc, I1))
        V0 = jnp.where(gt0, s, V0)
        I0 = jnp.where(gt0, cc, I0)
    v0[...], v1[...], v2[...] = V0, V1, V2
    i0[...], i1[...], i2[...] = I0, I1, I2

    @pl.when(t == _NSTEPS - 1)
    def _final():
        allv = jnp.concatenate([v0[...], v1[...], v2[...]], axis=1)
        allc = jnp.concatenate([i0[...], i1[...], i2[...]], axis=1)
        lane3 = jax.lax.broadcasted_iota(jnp.int32, (_B, 3 * _G), 1) & (_G - 1)
        alli = allc * _G + lane3
        av, ai = allv, alli
        for j in range(_K):
            m = jnp.max(av, axis=1, keepdims=True)
            hit = av == m
            cand = jnp.min(jnp.where(hit, ai, _BIGI), axis=1, keepdims=True)
            scores_ref[:, j:j + 1] = m
            idx_ref[:, j:j + 1] = cand
            av = jnp.where(hit & (ai == cand), _NEG, av)


def _knn_call(queries, database):
    return pl.pallas_call(
        _knn_body,
        grid=(_NSTEPS,),
        in_specs=[
            pl.BlockSpec((_B, _D), lambda t: (0, 0)),
            pl.BlockSpec((_C, _D + 1), lambda t: (t, 0)),
        ],
        out_specs=[
            pl.BlockSpec((_B, _K), lambda t: (0, 0)),
            pl.BlockSpec((_B, _K), lambda t: (0, 0)),
        ],
        out_shape=[
            jax.ShapeDtypeStruct((_B, _K), jnp.float32),
            jax.ShapeDtypeStruct((_B, _K), jnp.int32),
        ],
        scratch_shapes=[
            pltpu.VMEM((_B, _D), jnp.float32),
            pltpu.VMEM((_B, _G), jnp.float32),
            pltpu.VMEM((_B, _G), jnp.float32),
            pltpu.VMEM((_B, _G), jnp.float32),
            pltpu.VMEM((_B, _G), jnp.int32),
            pltpu.VMEM((_B, _G), jnp.int32),
            pltpu.VMEM((_B, _G), jnp.int32),
        ],
    )(queries, database)


_NW = 32
_BPW = (_B * _K) // _NW


def _gather_body(table_hbm, idx_hbm, rows_out, idx_v, rows_v, sem):
    wid = jax.lax.axis_index("s") * 2 + jax.lax.axis_index("c")
    base = wid * _BPW
    pltpu.sync_copy(idx_hbm.at[pl.ds(base, _BPW)], idx_v)
    pltpu.async_copy(table_hbm.at[idx_v], rows_v, sem).wait()
    pltpu.sync_copy(rows_v, rows_out.at[pl.ds(base, _BPW)])


def _gather_call(table, idx_flat):
    fn = pl.kernel(
        _gather_body,
        mesh=plsc.VectorSubcoreMesh(core_axis_name="c", subcore_axis_name="s"),
        out_type=jax.ShapeDtypeStruct((_B * _K, 128), jnp.float32),
        scratch_types=[
            pltpu.VMEM((_BPW,), jnp.int32),
            pltpu.VMEM((_BPW, 128), jnp.float32),
            pltpu.SemaphoreType.DMA,
        ],
    )
    return fn(table, idx_flat)


def _proj(x, w, b):
    y = jax.lax.dot_general(x, w, (((1,), (1,)), ((), ())),
                            preferred_element_type=jnp.float32)
    return y + b.reshape(1, -1)


def _mea_body(q_ref, rows_ref, sc_ref,
              WQ_ref, bQ_ref, WK_ref, bK_ref, WV_ref, bV_ref, WO_ref, bO_ref,
              dW_ref, db_ref, oW_ref, ob_ref, out_ref):
    q = q_ref[...]
    rows = rows_ref[:, :_D]
    lab = rows_ref[:, _D:_D + 1]
    sc = sc_ref[...]

    rn = rows / jnp.sqrt(jnp.sum(rows * rows, axis=1, keepdims=True))
    labi = lab.astype(jnp.int32)
    lane = jax.lax.broadcasted_iota(jnp.int32, (_B * _K, _D), 1)
    sc_km = jnp.concatenate([sc[:, k:k + 1] for k in range(_K)], axis=0)
    cls = jnp.where(lane == labi, sc_km, jnp.float32(0.0))
    hx = jnp.concatenate([q, q, q], axis=0)

    T = [cls, hx, rn]
    for i in range(_L):
        Q = [_proj(T[j], WQ_ref[i], bQ_ref[i]) for j in range(3)]
        Kt = [_proj(T[j], WK_ref[i], bK_ref[i]) for j in range(3)]
        V = [_proj(T[j], WV_ref[i], bV_ref[i]) for j in range(3)]
        newT = []
        for a in range(3):
            att = [jnp.sum(Q[a] * Kt[b], axis=1, keepdims=True) / _SCALE
                   for b in range(3)]
            m = jnp.maximum(jnp.maximum(att[0], att[1]), att[2])
            e = [jnp.exp(x - m) for x in att]
            den = e[0] + e[1] + e[2]
            o = (e[0] * V[0] + e[1] * V[1] + e[2] * V[2]) / den
            newT.append(_proj(o, WO_ref[i], bO_ref[i]))
        T = newT

    x = jnp.tanh(_proj(T[0], dW_ref[...], db_ref[...]))
    logits = _proj(x, oW_ref[...], ob_ref[...])
    lm = (logits[0:_B] + logits[_B:2 * _B] + logits[2 * _B:3 * _B]) / 3.0

    lane12 = jax.lax.broadcasted_iota(jnp.int32, (_B, _NUM_LABELS), 1)
    oh = jnp.zeros((_B, _NUM_LABELS), jnp.float32)
    for k in range(_K):
        oh = oh + jnp.where(lane12 == labi[k * _B:(k + 1) * _B], 1.0, 0.0)
    retr = oh / jnp.sum(oh, axis=1, keepdims=True)

    out_ref[...] = lm * (1.0 - _RATIO) + retr * _RATIO


def _mea_call(queries, rows128, scores,
              WQ, bQ, WK, bK, WV, bV, WO, bO, dense_W, dense_b, out_W, out_b):
    return pl.pallas_call(
        _mea_body,
        out_shape=jax.ShapeDtypeStruct((_B, _NUM_LABELS), jnp.float32),
    )(queries, rows128, scores,
      WQ, bQ, WK, bK, WV, bV, WO, bO, dense_W, dense_b, out_W, out_b)


def kernel(queries, database, WQ, bQ, WK, bK, WV, bV, WO, bO,
           dense_W, dense_b, out_W, out_b):
    scores, idxs = _knn_call(queries, database)
    idx_km = idxs.T.reshape(-1)
    table = jnp.pad(database, ((0, 0), (0, 128 - (_D + 1))))
    rows128 = _gather_call(table, idx_km)
    return _mea_call(queries, rows128, scores,
                     WQ, bQ, WK, bK, WV, bV, WO, bO,
                     dense_W, dense_b, out_W, out_b)
